# baseline (speedup 1.0000x reference)
import functools

import jax
import jax.numpy as jnp
from jax import lax
from jax.experimental import pallas as pl
from jax.experimental.pallas import tpu as pltpu
from jax.experimental.pallas import tpu_sc as plsc

N = 10000
E = 320000
D = 128
NC = 2
NS = 16
NW = NC * NS
EPW = E // NW
C = 80
NCHUNK = EPW // C
GROUPS = C // 16
ZR = 125
RPT = N // NS
S1D = 624
RS = 320
SUB = 80

f32 = jnp.float32
i32 = jnp.int32

_mesh = plsc.VectorSubcoreMesh(
    core_axis_name="c", subcore_axis_name="s", num_cores=NC, num_subcores=NS)


def _zero_rows(ref, nrows):
  def body(r, carry):
    for j in range(D // 16):
      ref[r, pl.ds(j * 16, 16)] = jnp.zeros((16,), f32)
    return carry
  lax.fori_loop(0, nrows, body, 0)


def _fill_1d(ref, n, value):
  def body(g, carry):
    ref[pl.ds(g * 16, 16)] = jnp.full((16,), value, f32)
    return carry
  lax.fori_loop(0, n // 16, body, 0)


def _zero_acc_2d(acc_s, zbuf, sid):
  for k in range(RPT // ZR):
    pltpu.sync_copy(zbuf, acc_s.at[pl.ds(sid * RPT + k * ZR, ZR)])


def _copy_1d_striped(src_ref, dst_ref, sid):
  pltpu.sync_copy(src_ref.at[pl.ds(sid * S1D, S1D)],
                  dst_ref.at[pl.ds(sid * S1D, S1D)])
  @pl.when(sid == NS - 1)
  def _():
    pltpu.sync_copy(src_ref.at[pl.ds(NS * S1D, N - NS * S1D)],
                    dst_ref.at[pl.ds(NS * S1D, N - NS * S1D)])


def _zero_1d_striped(zvec, dst_ref, sid):
  pltpu.sync_copy(zvec.at[pl.ds(0, S1D)], dst_ref.at[pl.ds(sid * S1D, S1D)])
  @pl.when(sid == NS - 1)
  def _():
    pltpu.sync_copy(zvec.at[pl.ds(0, N - NS * S1D)],
                    dst_ref.at[pl.ds(NS * S1D, N - NS * S1D)])


def _k1_body(x_ref, wt_ref, a_ref, h_ref, s12_ref):
  h = jnp.dot(x_ref[...], wt_ref[...], preferred_element_type=f32)
  h_ref[...] = h
  s12_ref[...] = jnp.dot(h, a_ref[...], preferred_element_type=f32)


def _k1(x, wt, a):
  blk = 1000
  grid = (N // blk,)
  return pl.pallas_call(
      _k1_body,
      grid=grid,
      in_specs=[
          pl.BlockSpec((blk, D), lambda i: (i, 0)),
          pl.BlockSpec((D, D), lambda i: (0, 0)),
          pl.BlockSpec((D, 2), lambda i: (0, 0)),
      ],
      out_specs=[
          pl.BlockSpec((blk, D), lambda i: (i, 0)),
          pl.BlockSpec((blk, 2), lambda i: (i, 0)),
      ],
      out_shape=[
          jax.ShapeDtypeStruct((N, D), f32),
          jax.ShapeDtypeStruct((N, 2), f32),
      ],
  )(x, wt, a)


def _zero_acc_from_rows(acc_s, rows_v, sid):
  for k in range(RPT // C):
    pltpu.sync_copy(rows_v, acc_s.at[pl.ds(sid * RPT + k * C, C)])
  rem = RPT - (RPT // C) * C
  if rem:
    pltpu.sync_copy(rows_v.at[pl.ds(0, rem)],
                    acc_s.at[pl.ds(sid * RPT + (RPT // C) * C, rem)])


NSLOT = 4


def _ka_body(s1, s2, ei, h, lu_out, den_out, deg_out,
             srcA, srcB, srcC, srcD, dstA, dstB, dstC, dstD,
             rowsA, rowsB, rowsC, rowsD,
             s1A, s1B, s1C, s1D, s2A, s2B, s2C, s2D,
             wA, wB, wC, wD, ones_v, zvec,
             acc_s, den_s, deg_s,
             isemA, isemB, isemC, isemD, gsemA, gsemB, gsemC, gsemD,
             ssemA, ssemB, ssemC, ssemD):
  cid = lax.axis_index("c")
  sid = lax.axis_index("s")
  wid = sid * NC + cid
  base = wid * EPW

  _zero_rows(rowsA, C)
  _fill_1d(zvec, S1D + 16, 0.0)
  _fill_1d(ones_v, C, 1.0)
  _zero_acc_from_rows(acc_s, rowsA, sid)
  _zero_1d_striped(zvec, den_s, sid)
  _zero_1d_striped(zvec, deg_s, sid)
  plsc.subcore_barrier()

  slots = ((srcA, dstA, rowsA, s1A, s2A, wA, isemA, gsemA, ssemA),
           (srcB, dstB, rowsB, s1B, s2B, wB, isemB, gsemB, ssemB),
           (srcC, dstC, rowsC, s1C, s2C, wC, isemC, gsemC, ssemC),
           (srcD, dstD, rowsD, s1D, s2D, wD, isemD, gsemD, ssemD))

  def phase1(c, k, slot):
    srcS, dstS, rowsS, s1S, s2S, wS, isem, gsem, ssem = slot
    @pl.when(k > 0)
    def _():
      pltpu.make_async_copy(rowsS, acc_s.at[dstS], ssem).wait()
      pltpu.make_async_copy(wS, den_s.at[dstS], ssem).wait()
      pltpu.make_async_copy(ones_v, deg_s.at[dstS], ssem).wait()
    off = base + c * C
    pltpu.async_copy(ei.at[0, pl.ds(off, C)], srcS, isem)
    pltpu.async_copy(ei.at[1, pl.ds(off, C)], dstS, isem)

  def phase2(c, slot):
    srcS, dstS, rowsS, s1S, s2S, wS, isem, gsem, ssem = slot
    off = base + c * C
    pltpu.make_async_copy(ei.at[0, pl.ds(off, C)], srcS, isem).wait()
    pltpu.make_async_copy(ei.at[1, pl.ds(off, C)], dstS, isem).wait()
    pltpu.async_copy(h.at[srcS], rowsS, gsem)
    pltpu.async_copy(s1.at[srcS], s1S, gsem)
    pltpu.async_copy(s2.at[dstS], s2S, gsem)

  def phase3(c, slot):
    srcS, dstS, rowsS, s1S, s2S, wS, isem, gsem, ssem = slot
    pltpu.make_async_copy(h.at[srcS], rowsS, gsem).wait()
    pltpu.make_async_copy(s1.at[srcS], s1S, gsem).wait()
    pltpu.make_async_copy(s2.at[dstS], s2S, gsem).wait()

    def group_body(g, gcarry):
      bq = g * 16
      e = s1S[pl.ds(bq, 16)] + s2S[pl.ds(bq, 16)]
      e = jnp.where(e >= 0.0, e, 0.2 * e)
      w = jnp.exp(e)
      wS[pl.ds(bq, 16)] = w
      dn = lax.GatherDimensionNumbers(
          offset_dims=(), collapsed_slice_dims=(0,), start_index_map=(0,))
      for i in range(16):
        r = bq + i
        wb = lax.gather(w, jnp.full((16, 1), i, i32), dn, (1,),
                        mode=lax.GatherScatterMode.PROMISE_IN_BOUNDS)
        for j in range(D // 16):
          rows_slice = rowsS[r, pl.ds(j * 16, 16)]
          rowsS[r, pl.ds(j * 16, 16)] = rows_slice * wb
      return gcarry
    lax.fori_loop(0, GROUPS, group_body, 0)

    pltpu.async_copy(rowsS, acc_s.at[dstS], ssem, add=True)
    pltpu.async_copy(wS, den_s.at[dstS], ssem, add=True)
    pltpu.async_copy(ones_v, deg_s.at[dstS], ssem, add=True)

  def body3(k, carry):
    c0 = NSLOT * k
    for s in range(NSLOT):
      @pl.when(c0 + s < NCHUNK)
      def _(s=s):
        phase1(c0 + s, k, slots[s])
    for s in range(NSLOT):
      @pl.when(c0 + s < NCHUNK)
      def _(s=s):
        phase2(c0 + s, slots[s])
    for s in range(NSLOT):
      @pl.when(c0 + s < NCHUNK)
      def _(s=s):
        phase3(c0 + s, slots[s])
    return carry
  lax.fori_loop(0, (NCHUNK + NSLOT - 1) // NSLOT, body3, 0)
  for s in range(NSLOT):
    srcS, dstS, rowsS, wS = slots[s][0], slots[s][1], slots[s][2], slots[s][5]
    pltpu.make_async_copy(rowsS, acc_s.at[dstS], slots[s][8]).wait()
    pltpu.make_async_copy(wS, den_s.at[dstS], slots[s][8]).wait()
    pltpu.make_async_copy(ones_v, deg_s.at[dstS], slots[s][8]).wait()

  plsc.subcore_barrier()
  for k in range(RPT // ZR):
    off = sid * RPT + k * ZR
    pltpu.sync_copy(acc_s.at[pl.ds(off, ZR)], lu_out.at[cid, pl.ds(off, ZR)])
  _copy_1d_striped(den_s, den_out.at[cid], sid)
  _copy_1d_striped(deg_s, deg_out.at[cid], sid)


_ka = functools.partial(
    pl.kernel,
    _ka_body,
    out_type=[
        jax.ShapeDtypeStruct((NC, N, D), f32),
        jax.ShapeDtypeStruct((NC, N), f32),
        jax.ShapeDtypeStruct((NC, N), f32),
    ],
    mesh=_mesh,
    compiler_params=pltpu.CompilerParams(use_tc_tiling_on_sc=False, needs_layout_passes=False),
    scratch_types=(
        [pltpu.VMEM((C,), i32) for _ in range(NSLOT)]
        + [pltpu.VMEM((C,), i32) for _ in range(NSLOT)]
        + [pltpu.VMEM((C, D), f32) for _ in range(NSLOT)]
        + [pltpu.VMEM((C,), f32) for _ in range(NSLOT)]
        + [pltpu.VMEM((C,), f32) for _ in range(NSLOT)]
        + [pltpu.VMEM((C,), f32) for _ in range(NSLOT)]
        + [
            pltpu.VMEM((C,), f32),
            pltpu.VMEM((S1D + 16,), f32),
            pltpu.VMEM_SHARED((N, D), f32),
            pltpu.VMEM_SHARED((N,), f32),
            pltpu.VMEM_SHARED((N,), f32),
        ]
        + [pltpu.SemaphoreType.DMA for _ in range(3 * NSLOT)]
    ),
)()


CKB = 128
KB_SLOTS = 3


def _kb_body(ei, table, tok, g_out,
             srcA, srcB, srcC, dstA, dstB, dstC, rowsA, rowsB, rowsC, acc_s,
             isemA, isemB, isemC, gsemA, gsemB, gsemC, ssemA, ssemB, ssemC):
  cid = lax.axis_index("c")
  sid = lax.axis_index("s")
  wid = sid * NC + cid
  nch = jnp.where(wid < 4, 79, 78)
  base = CKB * (78 * wid + jnp.minimum(wid, 4))

  _zero_rows(rowsA, CKB)
  _zero_acc_from_rows_kb(acc_s, rowsA, sid)
  plsc.subcore_barrier()

  slots = ((srcA, dstA, rowsA, isemA, gsemA, ssemA),
           (srcB, dstB, rowsB, isemB, gsemB, ssemB),
           (srcC, dstC, rowsC, isemC, gsemC, ssemC))

  def phase1(c, k, slot):
    srcS, dstS, rowsS, isem, gsem, ssem = slot
    @pl.when(k > 0)
    def _():
      pltpu.make_async_copy(rowsS, acc_s.at[dstS], ssem).wait()
    off = base + c * CKB
    pltpu.async_copy(ei.at[0, pl.ds(off, CKB)], srcS, isem)
    pltpu.async_copy(ei.at[1, pl.ds(off, CKB)], dstS, isem)

  def phase2(c, slot):
    srcS, dstS, rowsS, isem, gsem, ssem = slot
    off = base + c * CKB
    pltpu.make_async_copy(ei.at[0, pl.ds(off, CKB)], srcS, isem).wait()
    pltpu.make_async_copy(ei.at[1, pl.ds(off, CKB)], dstS, isem).wait()
    pltpu.async_copy(table.at[srcS], rowsS, gsem)

  def phase3(c, slot):
    srcS, dstS, rowsS, isem, gsem, ssem = slot
    pltpu.make_async_copy(table.at[srcS], rowsS, gsem).wait()
    pltpu.async_copy(rowsS, acc_s.at[dstS], ssem, add=True)

  def body3(k, carry):
    c0 = KB_SLOTS * k
    for s in range(KB_SLOTS):
      @pl.when(c0 + s < nch)
      def _(s=s):
        phase1(c0 + s, k, slots[s])
    for s in range(KB_SLOTS):
      @pl.when(c0 + s < nch)
      def _(s=s):
        phase2(c0 + s, slots[s])
    for s in range(KB_SLOTS):
      @pl.when(c0 + s < nch)
      def _(s=s):
        phase3(c0 + s, slots[s])
    return carry
  lax.fori_loop(0, (79 + KB_SLOTS - 1) // KB_SLOTS, body3, 0)
  for s in range(KB_SLOTS):
    srcS, dstS, rowsS = slots[s][0], slots[s][1], slots[s][2]
    pltpu.make_async_copy(rowsS, acc_s.at[dstS], slots[s][5]).wait()

  plsc.subcore_barrier()
  for k in range(RPT // ZR):
    off = sid * RPT + k * ZR
    pltpu.sync_copy(acc_s.at[pl.ds(off, ZR)], g_out.at[cid, pl.ds(off, ZR)])


def _zero_acc_from_rows_kb(acc_s, rows_v, sid):
  for k in range(RPT // CKB):
    pltpu.sync_copy(rows_v, acc_s.at[pl.ds(sid * RPT + k * CKB, CKB)])
  rem = RPT - (RPT // CKB) * CKB
  if rem:
    pltpu.sync_copy(rows_v.at[pl.ds(0, rem)],
                    acc_s.at[pl.ds(sid * RPT + (RPT // CKB) * CKB, rem)])


_kb = functools.partial(
    pl.kernel,
    _kb_body,
    out_type=jax.ShapeDtypeStruct((NC, N, D), f32),
    mesh=_mesh,
    compiler_params=pltpu.CompilerParams(use_tc_tiling_on_sc=False, needs_layout_passes=False),
    scratch_types=(
        [pltpu.VMEM((CKB,), i32) for _ in range(KB_SLOTS)]
        + [pltpu.VMEM((CKB,), i32) for _ in range(KB_SLOTS)]
        + [pltpu.VMEM((CKB, D), f32) for _ in range(KB_SLOTS)]
        + [pltpu.VMEM_SHARED((N, D), f32)]
        + [pltpu.SemaphoreType.DMA for _ in range(3 * KB_SLOTS)]
    ),
)()


def _kc_body(g0_ref, g1_ref, degt_ref, out_ref):
  dg = degt_ref[...]
  invdeg = 1.0 / jnp.maximum(dg[:, 0:1] + dg[:, 1:2], 1.0)
  out_ref[...] = (g0_ref[...] + g1_ref[...]) * invdeg


def _kc(g0, g1, degt):
  blk = 1000
  grid = (N // blk,)
  big = pl.BlockSpec((blk, D), lambda i: (i, 0))
  two = pl.BlockSpec((blk, 2), lambda i: (i, 0))
  return pl.pallas_call(
      _kc_body,
      grid=grid,
      in_specs=[big, big, two],
      out_specs=big,
      out_shape=jax.ShapeDtypeStruct((N, D), f32),
  )(g0, g1, degt)


def _kf_body(lu0_ref, lu1_ref, dent_ref, g2u0_ref, g2u1_ref, degt_ref,
             gftwt_ref, w1t_ref, w2t_ref, gb_ref, bb_ref, out_ref):
  dn = dent_ref[...]
  invden = 1.0 / (dn[:, 0:1] + dn[:, 1:2] + 1e-16)
  dg = degt_ref[...]
  invdeg = 1.0 / jnp.maximum(dg[:, 0:1] + dg[:, 1:2], 1.0)
  lu = (lu0_ref[...] + lu1_ref[...]) * invden
  local = jnp.where(lu > 0.0, lu, jnp.exp(jnp.minimum(lu, 0.0)) - 1.0)
  g2 = (g2u0_ref[...] + g2u1_ref[...]) * invdeg
  gf = jnp.dot(g2, gftwt_ref[...], preferred_element_type=f32) + gb_ref[...]
  acc = jnp.dot(local, w1t_ref[...], preferred_element_type=f32)
  acc = acc + jnp.dot(gf, w2t_ref[...], preferred_element_type=f32)
  out_ref[...] = jnp.maximum(acc + bb_ref[...], 0.0)


def _kf(lu0, lu1, dent, g2u0, g2u1, degt, gftwt, w1t, w2t, gb, bb):
  blk = 1000
  grid = (N // blk,)
  big = pl.BlockSpec((blk, D), lambda i: (i, 0))
  two = pl.BlockSpec((blk, 2), lambda i: (i, 0))
  wgt = pl.BlockSpec((D, D), lambda i: (0, 0))
  row = pl.BlockSpec((1, D), lambda i: (0, 0))
  return pl.pallas_call(
      _kf_body,
      grid=grid,
      in_specs=[big, big, two, big, big, two, wgt, wgt, wgt, row, row],
      out_specs=big,
      out_shape=jax.ShapeDtypeStruct((N, D), f32),
  )(lu0, lu1, dent, g2u0, g2u1, degt, gftwt, w1t, w2t, gb, bb)


@jax.jit
def kernel(node_features, edge_index, linear_weights, attention_weights,
           wt_W, wt_b, gft_W, gft_b):
  wt = linear_weights.T
  a = jnp.reshape(attention_weights, (2, D)).T
  h, s12 = _k1(node_features, wt, a)

  lu, den, deg = _ka(s12[:, 0], s12[:, 1], edge_index, h)
  g1u = _kb(edge_index, node_features, s12[:1, 0])
  degt = deg.T
  dent = den.T
  g1 = _kc(g1u[0], g1u[1], degt)
  g2u = _kb(edge_index, g1, s12[:1, 0])

  out = _kf(lu[0], lu[1], dent, g2u[0], g2u[1], degt,
            gft_W.T, wt_W[:, :D].T, wt_W[:, D:].T,
            gft_b.reshape(1, D), wt_b.reshape(1, D))
  return out


# speedup vs baseline: 1.0243x; 1.0243x over previous
import functools

import jax
import jax.numpy as jnp
from jax import lax
from jax.experimental import pallas as pl
from jax.experimental.pallas import tpu as pltpu
from jax.experimental.pallas import tpu_sc as plsc

N = 10000
E = 320000
D = 128
NC = 2
NS = 16
NW = NC * NS
EPW = E // NW
C = 80
NCHUNK = EPW // C
GROUPS = C // 16
ZR = 125
RPT = N // NS
S1D = 624
RS = 320
SUB = 80

f32 = jnp.float32
i32 = jnp.int32

_mesh = plsc.VectorSubcoreMesh(
    core_axis_name="c", subcore_axis_name="s", num_cores=NC, num_subcores=NS)


def _zero_rows(ref, nrows):
  def body(r, carry):
    for j in range(D // 16):
      ref[r, pl.ds(j * 16, 16)] = jnp.zeros((16,), f32)
    return carry
  lax.fori_loop(0, nrows, body, 0)


def _fill_1d(ref, n, value):
  def body(g, carry):
    ref[pl.ds(g * 16, 16)] = jnp.full((16,), value, f32)
    return carry
  lax.fori_loop(0, n // 16, body, 0)


def _zero_acc_2d(acc_s, zbuf, sid):
  for k in range(RPT // ZR):
    pltpu.sync_copy(zbuf, acc_s.at[pl.ds(sid * RPT + k * ZR, ZR)])


def _copy_1d_striped(src_ref, dst_ref, sid):
  pltpu.sync_copy(src_ref.at[pl.ds(sid * S1D, S1D)],
                  dst_ref.at[pl.ds(sid * S1D, S1D)])
  @pl.when(sid == NS - 1)
  def _():
    pltpu.sync_copy(src_ref.at[pl.ds(NS * S1D, N - NS * S1D)],
                    dst_ref.at[pl.ds(NS * S1D, N - NS * S1D)])


def _zero_1d_striped(zvec, dst_ref, sid):
  pltpu.sync_copy(zvec.at[pl.ds(0, S1D)], dst_ref.at[pl.ds(sid * S1D, S1D)])
  @pl.when(sid == NS - 1)
  def _():
    pltpu.sync_copy(zvec.at[pl.ds(0, N - NS * S1D)],
                    dst_ref.at[pl.ds(NS * S1D, N - NS * S1D)])


def _k1_body(x_ref, wt_ref, a_ref, h_ref, s12_ref):
  h = jnp.dot(x_ref[...], wt_ref[...], preferred_element_type=f32)
  h_ref[...] = h
  s12_ref[...] = jnp.dot(h, a_ref[...], preferred_element_type=f32)


def _k1(x, wt, a):
  blk = 1000
  grid = (N // blk,)
  return pl.pallas_call(
      _k1_body,
      grid=grid,
      in_specs=[
          pl.BlockSpec((blk, D), lambda i: (i, 0)),
          pl.BlockSpec((D, D), lambda i: (0, 0)),
          pl.BlockSpec((D, 2), lambda i: (0, 0)),
      ],
      out_specs=[
          pl.BlockSpec((blk, D), lambda i: (i, 0)),
          pl.BlockSpec((blk, 2), lambda i: (i, 0)),
      ],
      out_shape=[
          jax.ShapeDtypeStruct((N, D), f32),
          jax.ShapeDtypeStruct((N, 2), f32),
      ],
  )(x, wt, a)


def _zero_acc_from_rows(acc_s, rows_v, sid):
  for k in range(RPT // C):
    pltpu.sync_copy(rows_v, acc_s.at[pl.ds(sid * RPT + k * C, C)])
  rem = RPT - (RPT // C) * C
  if rem:
    pltpu.sync_copy(rows_v.at[pl.ds(0, rem)],
                    acc_s.at[pl.ds(sid * RPT + (RPT // C) * C, rem)])


NSLOT = 4


def _ka_body(s1, s2, ei, h, lu_out, den_out, deg_out,
             srcA, srcB, srcC, srcD, dstA, dstB, dstC, dstD,
             rowsA, rowsB, rowsC, rowsD,
             s1A, s1B, s1C, s1D, s2A, s2B, s2C, s2D,
             wA, wB, wC, wD, ones_v, zvec,
             acc_s, den_s, deg_s,
             isemA, isemB, isemC, isemD, gsemA, gsemB, gsemC, gsemD,
             ssemA, ssemB, ssemC, ssemD):
  cid = lax.axis_index("c")
  sid = lax.axis_index("s")
  wid = sid * NC + cid
  base = wid * EPW

  _zero_rows(rowsA, C)
  _fill_1d(zvec, S1D + 16, 0.0)
  _fill_1d(ones_v, C, 1.0)
  _zero_acc_from_rows(acc_s, rowsA, sid)
  _zero_1d_striped(zvec, den_s, sid)
  _zero_1d_striped(zvec, deg_s, sid)
  plsc.subcore_barrier()

  slots = ((srcA, dstA, rowsA, s1A, s2A, wA, isemA, gsemA, ssemA),
           (srcB, dstB, rowsB, s1B, s2B, wB, isemB, gsemB, ssemB),
           (srcC, dstC, rowsC, s1C, s2C, wC, isemC, gsemC, ssemC),
           (srcD, dstD, rowsD, s1D, s2D, wD, isemD, gsemD, ssemD))

  def phase1(c, k, slot):
    srcS, dstS, rowsS, s1S, s2S, wS, isem, gsem, ssem = slot
    @pl.when(k > 0)
    def _():
      pltpu.make_async_copy(rowsS, acc_s.at[dstS], ssem).wait()
      pltpu.make_async_copy(wS, den_s.at[dstS], ssem).wait()
      pltpu.make_async_copy(ones_v, deg_s.at[dstS], ssem).wait()
    off = base + c * C
    pltpu.async_copy(ei.at[0, pl.ds(off, C)], srcS, isem)
    pltpu.async_copy(ei.at[1, pl.ds(off, C)], dstS, isem)

  def phase2(c, slot):
    srcS, dstS, rowsS, s1S, s2S, wS, isem, gsem, ssem = slot
    off = base + c * C
    pltpu.make_async_copy(ei.at[0, pl.ds(off, C)], srcS, isem).wait()
    pltpu.make_async_copy(ei.at[1, pl.ds(off, C)], dstS, isem).wait()
    pltpu.async_copy(h.at[srcS], rowsS, gsem)
    pltpu.async_copy(s1.at[srcS], s1S, gsem)
    pltpu.async_copy(s2.at[dstS], s2S, gsem)

  def phase3(c, slot):
    srcS, dstS, rowsS, s1S, s2S, wS, isem, gsem, ssem = slot
    pltpu.make_async_copy(h.at[srcS], rowsS, gsem).wait()
    pltpu.make_async_copy(s1.at[srcS], s1S, gsem).wait()
    pltpu.make_async_copy(s2.at[dstS], s2S, gsem).wait()

    def group_body(g, gcarry):
      bq = g * 16
      e = s1S[pl.ds(bq, 16)] + s2S[pl.ds(bq, 16)]
      e = jnp.where(e >= 0.0, e, 0.2 * e)
      w = jnp.exp(e)
      wS[pl.ds(bq, 16)] = w
      dn = lax.GatherDimensionNumbers(
          offset_dims=(), collapsed_slice_dims=(0,), start_index_map=(0,))
      for i in range(16):
        r = bq + i
        wb = lax.gather(w, jnp.full((16, 1), i, i32), dn, (1,),
                        mode=lax.GatherScatterMode.PROMISE_IN_BOUNDS)
        for j in range(D // 16):
          rows_slice = rowsS[r, pl.ds(j * 16, 16)]
          rowsS[r, pl.ds(j * 16, 16)] = rows_slice * wb
      return gcarry
    lax.fori_loop(0, GROUPS, group_body, 0)

    pltpu.async_copy(rowsS, acc_s.at[dstS], ssem, add=True)
    pltpu.async_copy(wS, den_s.at[dstS], ssem, add=True)
    pltpu.async_copy(ones_v, deg_s.at[dstS], ssem, add=True)

  def body3(k, carry):
    c0 = NSLOT * k
    for s in range(NSLOT):
      @pl.when(c0 + s < NCHUNK)
      def _(s=s):
        phase1(c0 + s, k, slots[s])
    for s in range(NSLOT):
      @pl.when(c0 + s < NCHUNK)
      def _(s=s):
        phase2(c0 + s, slots[s])
    for s in range(NSLOT):
      @pl.when(c0 + s < NCHUNK)
      def _(s=s):
        phase3(c0 + s, slots[s])
    return carry
  lax.fori_loop(0, (NCHUNK + NSLOT - 1) // NSLOT, body3, 0)
  for s in range(NSLOT):
    srcS, dstS, rowsS, wS = slots[s][0], slots[s][1], slots[s][2], slots[s][5]
    pltpu.make_async_copy(rowsS, acc_s.at[dstS], slots[s][8]).wait()
    pltpu.make_async_copy(wS, den_s.at[dstS], slots[s][8]).wait()
    pltpu.make_async_copy(ones_v, deg_s.at[dstS], slots[s][8]).wait()

  plsc.subcore_barrier()
  for k in range(RPT // ZR):
    off = sid * RPT + k * ZR
    pltpu.sync_copy(acc_s.at[pl.ds(off, ZR)], lu_out.at[cid, pl.ds(off, ZR)])
  _copy_1d_striped(den_s, den_out.at[cid], sid)
  _copy_1d_striped(deg_s, deg_out.at[cid], sid)


_ka = functools.partial(
    pl.kernel,
    _ka_body,
    out_type=[
        jax.ShapeDtypeStruct((NC, N, D), f32),
        jax.ShapeDtypeStruct((NC, N), f32),
        jax.ShapeDtypeStruct((NC, N), f32),
    ],
    mesh=_mesh,
    compiler_params=pltpu.CompilerParams(use_tc_tiling_on_sc=False, needs_layout_passes=False),
    scratch_types=(
        [pltpu.VMEM((C,), i32) for _ in range(NSLOT)]
        + [pltpu.VMEM((C,), i32) for _ in range(NSLOT)]
        + [pltpu.VMEM((C, D), f32) for _ in range(NSLOT)]
        + [pltpu.VMEM((C,), f32) for _ in range(NSLOT)]
        + [pltpu.VMEM((C,), f32) for _ in range(NSLOT)]
        + [pltpu.VMEM((C,), f32) for _ in range(NSLOT)]
        + [
            pltpu.VMEM((C,), f32),
            pltpu.VMEM((S1D + 16,), f32),
            pltpu.VMEM_SHARED((N, D), f32),
            pltpu.VMEM_SHARED((N,), f32),
            pltpu.VMEM_SHARED((N,), f32),
        ]
        + [pltpu.SemaphoreType.DMA for _ in range(3 * NSLOT)]
    ),
)()


def _kb_body(ei, table, tok, g_out,
             srcA, srcB, srcC, srcD, dstA, dstB, dstC, dstD,
             rowsA, rowsB, rowsC, rowsD, acc_s,
             isemA, isemB, isemC, isemD, gsemA, gsemB, gsemC, gsemD,
             ssemA, ssemB, ssemC, ssemD):
  cid = lax.axis_index("c")
  sid = lax.axis_index("s")
  wid = sid * NC + cid
  base = wid * EPW

  _zero_rows(rowsA, C)
  _zero_acc_from_rows(acc_s, rowsA, sid)
  plsc.subcore_barrier()

  slots = ((srcA, dstA, rowsA, isemA, gsemA, ssemA),
           (srcB, dstB, rowsB, isemB, gsemB, ssemB),
           (srcC, dstC, rowsC, isemC, gsemC, ssemC),
           (srcD, dstD, rowsD, isemD, gsemD, ssemD))

  def phase1(c, k, slot):
    srcS, dstS, rowsS, isem, gsem, ssem = slot
    @pl.when(k > 0)
    def _():
      pltpu.make_async_copy(rowsS, acc_s.at[dstS], ssem).wait()
    off = base + c * C
    pltpu.async_copy(ei.at[0, pl.ds(off, C)], srcS, isem)
    pltpu.async_copy(ei.at[1, pl.ds(off, C)], dstS, isem)

  def phase2(c, slot):
    srcS, dstS, rowsS, isem, gsem, ssem = slot
    off = base + c * C
    pltpu.make_async_copy(ei.at[0, pl.ds(off, C)], srcS, isem).wait()
    pltpu.make_async_copy(ei.at[1, pl.ds(off, C)], dstS, isem).wait()
    pltpu.async_copy(table.at[srcS], rowsS, gsem)

  def phase3(c, slot):
    srcS, dstS, rowsS, isem, gsem, ssem = slot
    pltpu.make_async_copy(table.at[srcS], rowsS, gsem).wait()
    pltpu.async_copy(rowsS, acc_s.at[dstS], ssem, add=True)

  def body3(k, carry):
    c0 = NSLOT * k
    for s in range(NSLOT):
      @pl.when(c0 + s < NCHUNK)
      def _(s=s):
        phase1(c0 + s, k, slots[s])
    for s in range(NSLOT):
      @pl.when(c0 + s < NCHUNK)
      def _(s=s):
        phase2(c0 + s, slots[s])
    for s in range(NSLOT):
      @pl.when(c0 + s < NCHUNK)
      def _(s=s):
        phase3(c0 + s, slots[s])
    return carry
  lax.fori_loop(0, (NCHUNK + NSLOT - 1) // NSLOT, body3, 0)
  for s in range(NSLOT):
    srcS, dstS, rowsS = slots[s][0], slots[s][1], slots[s][2]
    pltpu.make_async_copy(rowsS, acc_s.at[dstS], slots[s][5]).wait()

  plsc.subcore_barrier()
  for k in range(RPT // ZR):
    off = sid * RPT + k * ZR
    pltpu.sync_copy(acc_s.at[pl.ds(off, ZR)], g_out.at[cid, pl.ds(off, ZR)])


_kb = functools.partial(
    pl.kernel,
    _kb_body,
    out_type=jax.ShapeDtypeStruct((NC, N, D), f32),
    mesh=_mesh,
    compiler_params=pltpu.CompilerParams(use_tc_tiling_on_sc=False, needs_layout_passes=False),
    scratch_types=(
        [pltpu.VMEM((C,), i32) for _ in range(NSLOT)]
        + [pltpu.VMEM((C,), i32) for _ in range(NSLOT)]
        + [pltpu.VMEM((C, D), f32) for _ in range(NSLOT)]
        + [pltpu.VMEM_SHARED((N, D), f32)]
        + [pltpu.SemaphoreType.DMA for _ in range(3 * NSLOT)]
    ),
)()


def _kc_body(g0_ref, g1_ref, degt_ref, out_ref):
  dg = degt_ref[...]
  invdeg = 1.0 / jnp.maximum(dg[:, 0:1] + dg[:, 1:2], 1.0)
  out_ref[...] = (g0_ref[...] + g1_ref[...]) * invdeg


def _kc(g0, g1, degt):
  blk = 1000
  grid = (N // blk,)
  big = pl.BlockSpec((blk, D), lambda i: (i, 0))
  two = pl.BlockSpec((blk, 2), lambda i: (i, 0))
  return pl.pallas_call(
      _kc_body,
      grid=grid,
      in_specs=[big, big, two],
      out_specs=big,
      out_shape=jax.ShapeDtypeStruct((N, D), f32),
  )(g0, g1, degt)


def _kf_body(lu0_ref, lu1_ref, dent_ref, g2u0_ref, g2u1_ref, degt_ref,
             gftwt_ref, w1t_ref, w2t_ref, gb_ref, bb_ref, out_ref):
  dn = dent_ref[...]
  invden = 1.0 / (dn[:, 0:1] + dn[:, 1:2] + 1e-16)
  dg = degt_ref[...]
  invdeg = 1.0 / jnp.maximum(dg[:, 0:1] + dg[:, 1:2], 1.0)
  lu = (lu0_ref[...] + lu1_ref[...]) * invden
  local = jnp.where(lu > 0.0, lu, jnp.exp(jnp.minimum(lu, 0.0)) - 1.0)
  g2 = (g2u0_ref[...] + g2u1_ref[...]) * invdeg
  gf = jnp.dot(g2, gftwt_ref[...], preferred_element_type=f32) + gb_ref[...]
  acc = jnp.dot(local, w1t_ref[...], preferred_element_type=f32)
  acc = acc + jnp.dot(gf, w2t_ref[...], preferred_element_type=f32)
  out_ref[...] = jnp.maximum(acc + bb_ref[...], 0.0)


def _kf(lu0, lu1, dent, g2u0, g2u1, degt, gftwt, w1t, w2t, gb, bb):
  blk = 1000
  grid = (N // blk,)
  big = pl.BlockSpec((blk, D), lambda i: (i, 0))
  two = pl.BlockSpec((blk, 2), lambda i: (i, 0))
  wgt = pl.BlockSpec((D, D), lambda i: (0, 0))
  row = pl.BlockSpec((1, D), lambda i: (0, 0))
  return pl.pallas_call(
      _kf_body,
      grid=grid,
      in_specs=[big, big, two, big, big, two, wgt, wgt, wgt, row, row],
      out_specs=big,
      out_shape=jax.ShapeDtypeStruct((N, D), f32),
  )(lu0, lu1, dent, g2u0, g2u1, degt, gftwt, w1t, w2t, gb, bb)


@jax.jit
def kernel(node_features, edge_index, linear_weights, attention_weights,
           wt_W, wt_b, gft_W, gft_b):
  wt = linear_weights.T
  a = jnp.reshape(attention_weights, (2, D)).T
  h, s12 = _k1(node_features, wt, a)

  lu, den, deg = _ka(s12[:, 0], s12[:, 1], edge_index, h)
  g1u = _kb(edge_index, node_features, s12[:1, 0])
  degt = deg.T
  dent = den.T
  g1 = _kc(g1u[0], g1u[1], degt)
  g2u = _kb(edge_index, g1, s12[:1, 0])

  out = _kf(lu[0], lu[1], dent, g2u[0], g2u[1], degt,
            gft_W.T, wt_W[:, :D].T, wt_W[:, D:].T,
            gft_b.reshape(1, D), wt_b.reshape(1, D))
  return out
